# speedup vs baseline: 3.0606x; 3.0606x over previous
import functools

import jax
import jax.numpy as jnp
from jax import lax
from jax.experimental import pallas as pl
from jax.experimental.pallas import tpu as pltpu
from jax.experimental.pallas import tpu_sc as plsc

N_NODES = 10000
D = 128
E = 320000
ITRS = 5

NW = 32
CHUNK = 128
NCH = 80
EPW = NCH * CHUNK
E_PAD = NW * EPW
N_PAD = 10240
RPS = N_PAD // 16


_mesh = plsc.VectorSubcoreMesh(core_axis_name="c", subcore_axis_name="s")


@functools.partial(
    pl.kernel,
    out_type=(
        jax.ShapeDtypeStruct((N_PAD, D), jnp.float32),
        jax.ShapeDtypeStruct((N_PAD, D), jnp.float32),
    ),
    mesh=_mesh,
    scratch_types=[
        pltpu.VMEM((NCH, CHUNK), jnp.int32),
        pltpu.VMEM((NCH, CHUNK), jnp.int32),
        pltpu.VMEM((CHUNK, D), jnp.float32),
        pltpu.VMEM_SHARED((N_PAD, D), jnp.float32),
        pltpu.SemaphoreType.DMA,
    ],
)
def _sc_spmm(h_hbm, rows_hbm, cols_hbm, zeros_hbm, out0, out1,
             row_v, col_v, buf, acc, gsem):
    cid = lax.axis_index("c")
    sid = lax.axis_index("s")
    wid = sid * 2 + cid
    rslice = pl.ds(sid * RPS, RPS)

    @pl.when(cid == 0)
    def _():
        pltpu.sync_copy(h_hbm.at[rslice], acc.at[rslice])

    @pl.when(cid != 0)
    def _():
        pltpu.sync_copy(zeros_hbm.at[rslice], acc.at[rslice])

    pltpu.sync_copy(rows_hbm.at[wid], row_v)
    pltpu.sync_copy(cols_hbm.at[wid], col_v)
    plsc.subcore_barrier()

    def body(j, carry):
        pltpu.async_copy(h_hbm.at[col_v.at[j]], buf, gsem).wait()
        pltpu.sync_copy(buf, acc.at[row_v.at[j]], add=True)
        return carry

    lax.fori_loop(0, NCH, body, 0, unroll=False)

    plsc.subcore_barrier()

    @pl.when(cid == 0)
    def _():
        pltpu.sync_copy(acc.at[rslice], out0.at[rslice])

    @pl.when(cid != 0)
    def _():
        pltpu.sync_copy(acc.at[rslice], out1.at[rslice])


def _row_mask(u):
    rows = lax.broadcasted_iota(jnp.int32, (N_PAD, D), 0)
    return jnp.where(rows < N_NODES, u, 0.0)


def _colnorm(u):
    n = jnp.sqrt(jnp.sum(u * u, axis=0, keepdims=True))
    return u / jnp.maximum(n, 1e-12)


def _norm1_body(x_ref, o_ref):
    o_ref[...] = _colnorm(_row_mask(x_ref[...]))


def _norm2_body(a_ref, b_ref, o_ref):
    o_ref[...] = _colnorm(_row_mask(a_ref[...] + b_ref[...]))


_norm1 = pl.pallas_call(
    _norm1_body, out_shape=jax.ShapeDtypeStruct((N_PAD, D), jnp.float32))
_norm2 = pl.pallas_call(
    _norm2_body, out_shape=jax.ShapeDtypeStruct((N_PAD, D), jnp.float32))


def _mlp_body(h_ref, w1_ref, b1_ref, w2_ref, b2_ref, w3_ref, b3_ref, o_ref):
    t = h_ref[...]
    t = jnp.maximum(
        jnp.dot(t, w1_ref[...], preferred_element_type=jnp.float32)
        + b1_ref[...], 0.0)
    t = jnp.maximum(
        jnp.dot(t, w2_ref[...], preferred_element_type=jnp.float32)
        + b2_ref[...], 0.0)
    o_ref[...] = (
        jnp.dot(t, w3_ref[...], preferred_element_type=jnp.float32)
        + b3_ref[...])


_MLP_BLK = 1280
_w_spec = pl.BlockSpec((D, D), lambda i: (0, 0))
_b_spec = pl.BlockSpec((1, D), lambda i: (0, 0))
_mlp = pl.pallas_call(
    _mlp_body,
    grid=(N_PAD // _MLP_BLK,),
    in_specs=[pl.BlockSpec((_MLP_BLK, D), lambda i: (i, 0)),
              _w_spec, _b_spec, _w_spec, _b_spec, _w_spec, _b_spec],
    out_specs=pl.BlockSpec((_MLP_BLK, D), lambda i: (i, 0)),
    out_shape=jax.ShapeDtypeStruct((N_PAD, D), jnp.float32),
)


def kernel(x, edge_index, W1, b1, W2, b2, W3, b3):
    ei = edge_index.astype(jnp.int32)
    n_extra = E_PAD - E
    pad_rows = N_NODES + jnp.arange(n_extra, dtype=jnp.int32) % (N_PAD - N_NODES)
    rows = jnp.concatenate([ei[0], pad_rows]).reshape(NW, NCH, CHUNK)
    cols = jnp.concatenate(
        [ei[1], jnp.zeros((n_extra,), jnp.int32)]).reshape(NW, NCH, CHUNK)
    zeros = jnp.zeros((N_PAD, D), jnp.float32)

    h = _norm1(jnp.pad(x[0], ((0, N_PAD - N_NODES), (0, 0))))
    for _ in range(ITRS):
        a0, a1 = _sc_spmm(h, rows, cols, zeros)
        h = _norm2(a0, a1)
    out = _mlp(h, W1, b1.reshape(1, D), W2, b2.reshape(1, D),
               W3, b3.reshape(1, D))
    return out[:N_NODES][None]


# speedup vs baseline: 3.2303x; 1.0555x over previous
import functools

import jax
import jax.numpy as jnp
from jax import lax
from jax.experimental import pallas as pl
from jax.experimental.pallas import tpu as pltpu
from jax.experimental.pallas import tpu_sc as plsc

N_NODES = 10000
D = 128
E = 320000
ITRS = 5

NW = 32
CHUNK = 128
NCH = 80
EPW = NCH * CHUNK
E_PAD = NW * EPW
N_PAD = 10240
RPS = N_PAD // 16

NSLOT = 2
ISLOT = 4

_mesh = plsc.VectorSubcoreMesh(core_axis_name="c", subcore_axis_name="s")


@functools.partial(
    pl.kernel,
    out_type=(
        jax.ShapeDtypeStruct((N_PAD, D), jnp.float32),
        jax.ShapeDtypeStruct((N_PAD, D), jnp.float32),
    ),
    mesh=_mesh,
    scratch_types=[
        pltpu.VMEM((ISLOT, 2, CHUNK), jnp.int32),
        pltpu.VMEM((NSLOT * CHUNK, D), jnp.float32),
        pltpu.VMEM_SHARED((N_PAD, D), jnp.float32),
        [pltpu.SemaphoreType.DMA] * NSLOT,
        [pltpu.SemaphoreType.DMA] * NSLOT,
        [pltpu.SemaphoreType.DMA] * ISLOT,
    ],
)
def _sc_spmm(h_hbm, idx_hbm, zeros_hbm, out0, out1,
             idx_v, buf, acc, gsems, ssems, isems):
    cid = lax.axis_index("c")
    sid = lax.axis_index("s")
    wid = sid * 2 + cid
    rslice = pl.ds(sid * RPS, RPS)

    @pl.when(cid == 0)
    def _():
        pltpu.sync_copy(h_hbm.at[rslice], acc.at[rslice])

    @pl.when(cid != 0)
    def _():
        pltpu.sync_copy(zeros_hbm.at[rslice], acc.at[rslice])

    def bslot(b):
        return buf.at[pl.ds(b * CHUNK, CHUNK)]

    def load_idx(j, s):
        pltpu.async_copy(idx_hbm.at[wid, j], idx_v.at[s], isems[s])

    def wait_idx(j, s):
        pltpu.make_async_copy(idx_hbm.at[wid, j], idx_v.at[s],
                              isems[s]).wait()

    def issue_gather(j, b, s):
        pltpu.async_copy(h_hbm.at[idx_v.at[s, 1]], bslot(b), gsems[b])

    def wait_gather(j, b, s):
        pltpu.make_async_copy(h_hbm.at[idx_v.at[s, 1]], bslot(b),
                              gsems[b]).wait()

    def issue_scatter(j, b, s):
        pltpu.async_copy(bslot(b), acc.at[idx_v.at[s, 0]], ssems[b],
                         add=True)

    def wait_scatter(j, b, s):
        pltpu.make_async_copy(bslot(b), acc.at[idx_v.at[s, 0]],
                              ssems[b]).wait()

    def step(j, full):
        b, s = j % NSLOT, j % ISLOT
        wait_gather(j, b, s)
        issue_scatter(j, b, s)
        if j >= 1:
            wait_scatter(j - 1, 1 - b, (j - 1) % ISLOT)
        if full:
            load_idx(j + 3, (j + 3) % ISLOT)
        if j + 1 < NCH:
            wait_idx(j + 1, (j + 1) % ISLOT)
            issue_gather(j + 1, 1 - b, (j + 1) % ISLOT)

    for k in range(3):
        load_idx(k, k)
    plsc.subcore_barrier()
    wait_idx(0, 0)
    issue_gather(0, 0, 0)
    step(0, True)

    def body(g, carry):
        for i in range(4):
            j = 1 + g * 4 + i
            b, s = (1 + i) % NSLOT, (1 + i) % ISLOT
            wait_gather(j, b, s)
            issue_scatter(j, b, s)
            wait_scatter(j - 1, 1 - b, i % ISLOT)
            load_idx(j + 3, i % ISLOT)
            wait_idx(j + 1, (2 + i) % ISLOT)
            issue_gather(j + 1, 1 - b, (2 + i) % ISLOT)
        return carry

    lax.fori_loop(0, (NCH - 4) // 4, body, 0, unroll=False)
    for j in range(NCH - 3, NCH):
        step(j, False)
    wait_scatter(NCH - 1, (NCH - 1) % NSLOT, (NCH - 1) % ISLOT)

    plsc.subcore_barrier()

    @pl.when(cid == 0)
    def _():
        pltpu.sync_copy(acc.at[rslice], out0.at[rslice])

    @pl.when(cid != 0)
    def _():
        pltpu.sync_copy(acc.at[rslice], out1.at[rslice])


def _row_mask(u):
    rows = lax.broadcasted_iota(jnp.int32, (N_PAD, D), 0)
    return jnp.where(rows < N_NODES, u, 0.0)


def _colnorm(u):
    n = jnp.sqrt(jnp.sum(u * u, axis=0, keepdims=True))
    return u / jnp.maximum(n, 1e-12)


def _norm1_body(x_ref, o_ref):
    o_ref[...] = _colnorm(_row_mask(x_ref[...]))


def _norm2_body(a_ref, b_ref, o_ref):
    o_ref[...] = _colnorm(_row_mask(a_ref[...] + b_ref[...]))


_norm1 = pl.pallas_call(
    _norm1_body, out_shape=jax.ShapeDtypeStruct((N_PAD, D), jnp.float32))
_norm2 = pl.pallas_call(
    _norm2_body, out_shape=jax.ShapeDtypeStruct((N_PAD, D), jnp.float32))


def _mlp_body(h_ref, w1_ref, b1_ref, w2_ref, b2_ref, w3_ref, b3_ref, o_ref):
    t = h_ref[...]
    t = jnp.maximum(
        jnp.dot(t, w1_ref[...], preferred_element_type=jnp.float32)
        + b1_ref[...], 0.0)
    t = jnp.maximum(
        jnp.dot(t, w2_ref[...], preferred_element_type=jnp.float32)
        + b2_ref[...], 0.0)
    o_ref[...] = (
        jnp.dot(t, w3_ref[...], preferred_element_type=jnp.float32)
        + b3_ref[...])


_MLP_BLK = 1280
_w_spec = pl.BlockSpec((D, D), lambda i: (0, 0))
_b_spec = pl.BlockSpec((1, D), lambda i: (0, 0))
_mlp = pl.pallas_call(
    _mlp_body,
    grid=(N_PAD // _MLP_BLK,),
    in_specs=[pl.BlockSpec((_MLP_BLK, D), lambda i: (i, 0)),
              _w_spec, _b_spec, _w_spec, _b_spec, _w_spec, _b_spec],
    out_specs=pl.BlockSpec((_MLP_BLK, D), lambda i: (i, 0)),
    out_shape=jax.ShapeDtypeStruct((N_PAD, D), jnp.float32),
)


def kernel(x, edge_index, W1, b1, W2, b2, W3, b3):
    ei = edge_index.astype(jnp.int32)
    n_extra = E_PAD - E
    pad_rows = N_NODES + jnp.arange(n_extra, dtype=jnp.int32) % (N_PAD - N_NODES)
    rows = jnp.concatenate([ei[0], pad_rows]).reshape(NW, NCH, 1, CHUNK)
    cols = jnp.concatenate(
        [ei[1], jnp.zeros((n_extra,), jnp.int32)]).reshape(NW, NCH, 1, CHUNK)
    idx = jnp.concatenate([rows, cols], axis=2)
    zeros = jnp.zeros((N_PAD, D), jnp.float32)

    h = _norm1(jnp.pad(x[0], ((0, N_PAD - N_NODES), (0, 0))))
    for _ in range(ITRS):
        a0, a1 = _sc_spmm(h, idx, zeros)
        h = _norm2(a0, a1)
    out = _mlp(h, W1, b1.reshape(1, D), W2, b2.reshape(1, D),
               W3, b3.reshape(1, D))
    return out[:N_NODES][None]


# speedup vs baseline: 9.8495x; 3.0491x over previous
import functools

import jax
import jax.numpy as jnp
from jax import lax
from jax.experimental import pallas as pl
from jax.experimental.pallas import tpu as pltpu
from jax.experimental.pallas import tpu_sc as plsc

N_NODES = 10000
D = 128
E = 320000
ITRS = 5

NW = 32
CHUNK = 128
NCH = 80
EPW = NCH * CHUNK
E_PAD = NW * EPW
N_PAD = 10240
RPS = N_PAD // 16

NSLOT = 2
ISLOT = 4

_mesh = plsc.VectorSubcoreMesh(core_axis_name="c", subcore_axis_name="s")


@functools.partial(
    pl.kernel,
    out_type=(
        jax.ShapeDtypeStruct((N_PAD, D), jnp.float32),
        jax.ShapeDtypeStruct((N_PAD, D), jnp.float32),
    ),
    mesh=_mesh,
    scratch_types=[
        pltpu.VMEM((ISLOT, 2, CHUNK), jnp.int32),
        pltpu.VMEM((NSLOT * CHUNK, D), jnp.float32),
        pltpu.VMEM_SHARED((N_PAD, D), jnp.float32),
        [pltpu.SemaphoreType.DMA] * NSLOT,
        [pltpu.SemaphoreType.DMA] * NSLOT,
        [pltpu.SemaphoreType.DMA] * ISLOT,
    ],
)
def _sc_spmm(h_hbm, idx_hbm, zeros_hbm, out0, out1,
             idx_v, buf, acc, gsems, ssems, isems):
    cid = lax.axis_index("c")
    sid = lax.axis_index("s")
    wid = sid * 2 + cid
    rslice = pl.ds(sid * RPS, RPS)

    @pl.when(cid == 0)
    def _():
        pltpu.sync_copy(h_hbm.at[rslice], acc.at[rslice])

    @pl.when(cid != 0)
    def _():
        pltpu.sync_copy(zeros_hbm.at[rslice], acc.at[rslice])

    def bslot(b):
        return buf.at[pl.ds(b * CHUNK, CHUNK)]

    def load_idx(j, s):
        pltpu.async_copy(idx_hbm.at[wid, j], idx_v.at[s], isems[s])

    def wait_idx(j, s):
        pltpu.make_async_copy(idx_hbm.at[wid, j], idx_v.at[s],
                              isems[s]).wait()

    def issue_gather(j, b, s):
        pltpu.async_copy(h_hbm.at[idx_v.at[s, 1]], bslot(b), gsems[b])

    def wait_gather(j, b, s):
        pltpu.make_async_copy(h_hbm.at[idx_v.at[s, 1]], bslot(b),
                              gsems[b]).wait()

    def issue_scatter(j, b, s):
        pltpu.async_copy(bslot(b), acc.at[idx_v.at[s, 0]], ssems[b],
                         add=True)

    def wait_scatter(j, b, s):
        pltpu.make_async_copy(bslot(b), acc.at[idx_v.at[s, 0]],
                              ssems[b]).wait()

    def step(j, full):
        b, s = j % NSLOT, j % ISLOT
        wait_gather(j, b, s)
        issue_scatter(j, b, s)
        if j >= 1:
            wait_scatter(j - 1, 1 - b, (j - 1) % ISLOT)
        if full:
            load_idx(j + 3, (j + 3) % ISLOT)
        if j + 1 < NCH:
            wait_idx(j + 1, (j + 1) % ISLOT)
            issue_gather(j + 1, 1 - b, (j + 1) % ISLOT)

    for k in range(3):
        load_idx(k, k)
    plsc.subcore_barrier()
    wait_idx(0, 0)
    issue_gather(0, 0, 0)
    step(0, True)

    def body(g, carry):
        for i in range(4):
            j = 1 + g * 4 + i
            b, s = (1 + i) % NSLOT, (1 + i) % ISLOT
            wait_gather(j, b, s)
            issue_scatter(j, b, s)
            wait_scatter(j - 1, 1 - b, i % ISLOT)
            load_idx(j + 3, i % ISLOT)
            wait_idx(j + 1, (2 + i) % ISLOT)
            issue_gather(j + 1, 1 - b, (2 + i) % ISLOT)
        return carry

    lax.fori_loop(0, (NCH - 4) // 4, body, 0, unroll=False)
    for j in range(NCH - 3, NCH):
        step(j, False)
    wait_scatter(NCH - 1, (NCH - 1) % NSLOT, (NCH - 1) % ISLOT)

    plsc.subcore_barrier()

    @pl.when(cid == 0)
    def _():
        pltpu.sync_copy(acc.at[rslice], out0.at[rslice])

    @pl.when(cid != 0)
    def _():
        pltpu.sync_copy(acc.at[rslice], out1.at[rslice])


def _row_mask(u):
    rows = lax.broadcasted_iota(jnp.int32, (N_PAD, D), 0)
    return jnp.where(rows < N_NODES, u, 0.0)


def _colnorm(u):
    n = jnp.sqrt(jnp.sum(u * u, axis=0, keepdims=True))
    return u / jnp.maximum(n, 1e-12)


def _norm1_body(x_ref, o_ref):
    o_ref[...] = _colnorm(_row_mask(x_ref[...]))


def _norm2_body(a_ref, b_ref, o_ref):
    o_ref[...] = _colnorm(_row_mask(a_ref[...] + b_ref[...]))


_norm1 = pl.pallas_call(
    _norm1_body, out_shape=jax.ShapeDtypeStruct((N_PAD, D), jnp.float32))
_norm2 = pl.pallas_call(
    _norm2_body, out_shape=jax.ShapeDtypeStruct((N_PAD, D), jnp.float32))


def _mlp_body(h_ref, w1_ref, b1_ref, w2_ref, b2_ref, w3_ref, b3_ref, o_ref):
    t = h_ref[...]
    t = jnp.maximum(
        jnp.dot(t, w1_ref[...], preferred_element_type=jnp.float32)
        + b1_ref[...], 0.0)
    t = jnp.maximum(
        jnp.dot(t, w2_ref[...], preferred_element_type=jnp.float32)
        + b2_ref[...], 0.0)
    o_ref[...] = (
        jnp.dot(t, w3_ref[...], preferred_element_type=jnp.float32)
        + b3_ref[...])


_MLP_BLK = 1280
_w_spec = pl.BlockSpec((D, D), lambda i: (0, 0))
_b_spec = pl.BlockSpec((1, D), lambda i: (0, 0))
_mlp = pl.pallas_call(
    _mlp_body,
    grid=(N_PAD // _MLP_BLK,),
    in_specs=[pl.BlockSpec((_MLP_BLK, D), lambda i: (i, 0)),
              _w_spec, _b_spec, _w_spec, _b_spec, _w_spec, _b_spec],
    out_specs=pl.BlockSpec((_MLP_BLK, D), lambda i: (i, 0)),
    out_shape=jax.ShapeDtypeStruct((N_PAD, D), jnp.float32),
)


def kernel(x, edge_index, W1, b1, W2, b2, W3, b3):
    ei = edge_index.astype(jnp.int32)
    n_extra = E_PAD - E
    pad_iota = jnp.arange(n_extra, dtype=jnp.int32)
    pad_rows = N_NODES + pad_iota % (N_PAD - N_NODES)
    pad_cols = (pad_iota * 13) % N_NODES
    rows = jnp.concatenate([ei[0], pad_rows]).reshape(NW, NCH, 1, CHUNK)
    cols = jnp.concatenate([ei[1], pad_cols]).reshape(NW, NCH, 1, CHUNK)
    idx = jnp.concatenate([rows, cols], axis=2)
    zeros = jnp.zeros((N_PAD, D), jnp.float32)

    h = _norm1(jnp.pad(x[0], ((0, N_PAD - N_NODES), (0, 0))))
    for _ in range(ITRS):
        a0, a1 = _sc_spmm(h, idx, zeros)
        h = _norm2(a0, a1)
    out = _mlp(h, W1, b1.reshape(1, D), W2, b2.reshape(1, D),
               W3, b3.reshape(1, D))
    return out[:N_NODES][None]


# speedup vs baseline: 9.9943x; 1.0147x over previous
import functools

import jax
import jax.numpy as jnp
from jax import lax
from jax.experimental import pallas as pl
from jax.experimental.pallas import tpu as pltpu
from jax.experimental.pallas import tpu_sc as plsc

N_NODES = 10000
D = 128
E = 320000
ITRS = 5

NW = 32
CHUNK = 128
NCH = 80
EPW = NCH * CHUNK
E_PAD = NW * EPW
N_PAD = 10240
RPS = N_PAD // 16

NSLOT = 2
ISLOT = 4

_mesh = plsc.VectorSubcoreMesh(core_axis_name="c", subcore_axis_name="s")


@functools.partial(
    pl.kernel,
    out_type=(
        jax.ShapeDtypeStruct((N_PAD, D), jnp.float32),
        jax.ShapeDtypeStruct((N_PAD, D), jnp.float32),
    ),
    mesh=_mesh,
    scratch_types=[
        pltpu.VMEM((ISLOT, 2, CHUNK), jnp.int32),
        pltpu.VMEM((NSLOT * CHUNK, D), jnp.float32),
        pltpu.VMEM_SHARED((N_PAD, D), jnp.float32),
        [pltpu.SemaphoreType.DMA] * NSLOT,
        [pltpu.SemaphoreType.DMA] * NSLOT,
        [pltpu.SemaphoreType.DMA] * ISLOT,
    ],
)
def _sc_spmm(h_hbm, idx_hbm, zeros_hbm, out0, out1,
             idx_v, buf, acc, gsems, ssems, isems):
    cid = lax.axis_index("c")
    sid = lax.axis_index("s")
    wid = sid * 2 + cid
    rslice = pl.ds(sid * RPS, RPS)

    @pl.when(cid == 0)
    def _():
        pltpu.sync_copy(h_hbm.at[rslice], acc.at[rslice])

    @pl.when(cid != 0)
    def _():
        pltpu.sync_copy(zeros_hbm.at[rslice], acc.at[rslice])

    def bslot(b):
        return buf.at[pl.ds(b * CHUNK, CHUNK)]

    def load_idx(j, s):
        pltpu.async_copy(idx_hbm.at[wid, j], idx_v.at[s], isems[s])

    def wait_idx(j, s):
        pltpu.make_async_copy(idx_hbm.at[wid, j], idx_v.at[s],
                              isems[s]).wait()

    def issue_gather(j, b, s):
        pltpu.async_copy(h_hbm.at[idx_v.at[s, 1]], bslot(b), gsems[b])

    def wait_gather(j, b, s):
        pltpu.make_async_copy(h_hbm.at[idx_v.at[s, 1]], bslot(b),
                              gsems[b]).wait()

    def issue_scatter(j, b, s):
        pltpu.async_copy(bslot(b), acc.at[idx_v.at[s, 0]], ssems[b],
                         add=True)

    def wait_scatter(j, b, s):
        pltpu.make_async_copy(bslot(b), acc.at[idx_v.at[s, 0]],
                              ssems[b]).wait()

    def step(j, full):
        b, s = j % NSLOT, j % ISLOT
        wait_gather(j, b, s)
        issue_scatter(j, b, s)
        if j >= 1:
            wait_scatter(j - 1, 1 - b, (j - 1) % ISLOT)
        if full:
            load_idx(j + 3, (j + 3) % ISLOT)
        if j + 1 < NCH:
            wait_idx(j + 1, (j + 1) % ISLOT)
            issue_gather(j + 1, 1 - b, (j + 1) % ISLOT)

    for k in range(3):
        load_idx(k, k)
    plsc.subcore_barrier()
    wait_idx(0, 0)
    issue_gather(0, 0, 0)
    step(0, True)

    def body(g, carry):
        for i in range(4):
            j = 1 + g * 4 + i
            b, s = (1 + i) % NSLOT, (1 + i) % ISLOT
            wait_gather(j, b, s)
            issue_scatter(j, b, s)
            wait_scatter(j - 1, 1 - b, i % ISLOT)
            load_idx(j + 3, i % ISLOT)
            wait_idx(j + 1, (2 + i) % ISLOT)
            issue_gather(j + 1, 1 - b, (2 + i) % ISLOT)
        return carry

    lax.fori_loop(0, (NCH - 4) // 4, body, 0, unroll=False)
    for j in range(NCH - 3, NCH):
        step(j, False)
    wait_scatter(NCH - 1, (NCH - 1) % NSLOT, (NCH - 1) % ISLOT)

    plsc.subcore_barrier()

    @pl.when(cid == 0)
    def _():
        pltpu.sync_copy(acc.at[rslice], out0.at[rslice])

    @pl.when(cid != 0)
    def _():
        pltpu.sync_copy(acc.at[rslice], out1.at[rslice])


def _row_mask(u):
    rows = lax.broadcasted_iota(jnp.int32, (N_PAD, D), 0)
    return jnp.where(rows < N_NODES, u, 0.0)


def _colnorm(u):
    n = jnp.sqrt(jnp.sum(u * u, axis=0, keepdims=True))
    return u / jnp.maximum(n, 1e-12)


def _norm1_body(x_ref, o_ref):
    o_ref[...] = _colnorm(_row_mask(x_ref[...]))


def _norm2_body(a_ref, b_ref, o_ref):
    o_ref[...] = _colnorm(_row_mask(a_ref[...] + b_ref[...]))


_norm1 = pl.pallas_call(
    _norm1_body, out_shape=jax.ShapeDtypeStruct((N_PAD, D), jnp.float32))
_norm2 = pl.pallas_call(
    _norm2_body, out_shape=jax.ShapeDtypeStruct((N_PAD, D), jnp.float32))


def _final_body(a_ref, b_ref, w1_ref, b1_ref, w2_ref, b2_ref, w3_ref,
                b3_ref, o_ref):
    t = _colnorm(_row_mask(a_ref[...] + b_ref[...]))[:N_NODES]
    t = jnp.maximum(
        jnp.dot(t, w1_ref[...], preferred_element_type=jnp.float32)
        + b1_ref[...], 0.0)
    t = jnp.maximum(
        jnp.dot(t, w2_ref[...], preferred_element_type=jnp.float32)
        + b2_ref[...], 0.0)
    o_ref[...] = (
        jnp.dot(t, w3_ref[...], preferred_element_type=jnp.float32)
        + b3_ref[...])


_final = pl.pallas_call(
    _final_body,
    out_shape=jax.ShapeDtypeStruct((N_NODES, D), jnp.float32),
)


def kernel(x, edge_index, W1, b1, W2, b2, W3, b3):
    ei = edge_index.astype(jnp.int32)
    n_extra = E_PAD - E
    pad_iota = jnp.arange(n_extra, dtype=jnp.int32)
    pad_rows = N_NODES + pad_iota % (N_PAD - N_NODES)
    pad_cols = (pad_iota * 13) % N_NODES
    rows = jnp.concatenate([ei[0], pad_rows]).reshape(NW, NCH, 1, CHUNK)
    cols = jnp.concatenate([ei[1], pad_cols]).reshape(NW, NCH, 1, CHUNK)
    idx = jnp.concatenate([rows, cols], axis=2)
    zeros = jnp.zeros((N_PAD, D), jnp.float32)

    h = _norm1(jnp.pad(x[0], ((0, N_PAD - N_NODES), (0, 0))))
    for _ in range(ITRS - 1):
        a0, a1 = _sc_spmm(h, idx, zeros)
        h = _norm2(a0, a1)
    a0, a1 = _sc_spmm(h, idx, zeros)
    out = _final(a0, a1, W1, b1.reshape(1, D), W2, b2.reshape(1, D),
                 W3, b3.reshape(1, D))
    return out[None]


# speedup vs baseline: 11.7760x; 1.1783x over previous
import functools

import jax
import jax.numpy as jnp
from jax import lax
from jax.experimental import pallas as pl
from jax.experimental.pallas import tpu as pltpu
from jax.experimental.pallas import tpu_sc as plsc

N_NODES = 10000
D = 128
E = 320000
ITRS = 5

NW = 32
CHUNK = 128
NCH = 80
EPW = NCH * CHUNK
E_PAD = NW * EPW
N_PAD = 10240
RPS = N_PAD // 16

NSLOT = 2
ISLOT = 4

_mesh = plsc.VectorSubcoreMesh(core_axis_name="c", subcore_axis_name="s")


@functools.partial(
    pl.kernel,
    out_type=(
        jax.ShapeDtypeStruct((N_PAD, D), jnp.float32),
        jax.ShapeDtypeStruct((N_PAD, D), jnp.float32),
    ),
    mesh=_mesh,
    scratch_types=[
        pltpu.VMEM((ISLOT, 2, CHUNK), jnp.int32),
        pltpu.VMEM((NSLOT * CHUNK, D), jnp.float32),
        pltpu.VMEM_SHARED((N_PAD, D), jnp.float32),
        [pltpu.SemaphoreType.DMA] * NSLOT,
        [pltpu.SemaphoreType.DMA] * ISLOT,
    ],
)
def _sc_spmm(h_hbm, idx_hbm, zeros_hbm, out0, out1,
             idx_v, buf, acc, gsems, isems):
    cid = lax.axis_index("c")
    sid = lax.axis_index("s")
    wid = sid * 2 + cid
    rslice = pl.ds(sid * RPS, RPS)

    @pl.when(cid == 0)
    def _():
        pltpu.sync_copy(h_hbm.at[rslice], acc.at[rslice])

    @pl.when(cid != 0)
    def _():
        pltpu.sync_copy(zeros_hbm.at[rslice], acc.at[rslice])

    def bslot(b):
        return buf.at[pl.ds(b * CHUNK, CHUNK)]

    def load_idx(j, s):
        pltpu.async_copy(idx_hbm.at[wid, j], idx_v.at[s], isems[s])

    def wait_idx(j, s):
        pltpu.make_async_copy(idx_hbm.at[wid, j], idx_v.at[s],
                              isems[s]).wait()

    def issue_gather(j, b, s):
        pltpu.async_copy(h_hbm.at[idx_v.at[s, 1]], bslot(b), gsems[b])

    def wait_gather(j, b, s):
        pltpu.make_async_copy(h_hbm.at[idx_v.at[s, 1]], bslot(b),
                              gsems[b]).wait()

    def sync_scatter(j, b, s):
        pltpu.sync_copy(bslot(b), acc.at[idx_v.at[s, 0]], add=True)

    def step(j):
        b, s = j % NSLOT, j % ISLOT
        wait_gather(j, b, s)
        sync_scatter(j, b, s)
        if j + 3 < NCH:
            load_idx(j + 3, (j + 3) % ISLOT)
        if j + 2 < NCH:
            wait_idx(j + 2, (j + 2) % ISLOT)
            issue_gather(j + 2, b, (j + 2) % ISLOT)

    for k in range(3):
        load_idx(k, k)
    plsc.subcore_barrier()
    for k in (0, 1):
        wait_idx(k, k)
        issue_gather(k, k, k)
    step(0)

    def body(g, carry):
        for i in range(4):
            j = 1 + g * 4 + i
            b, s = (1 + i) % NSLOT, (1 + i) % ISLOT
            wait_gather(j, b, s)
            sync_scatter(j, b, s)
            load_idx(j + 3, i % ISLOT)
            wait_idx(j + 2, (3 + i) % ISLOT)
            issue_gather(j + 2, b, (3 + i) % ISLOT)
        return carry

    lax.fori_loop(0, (NCH - 4) // 4, body, 0, unroll=False)
    for j in range(NCH - 3, NCH):
        step(j)

    plsc.subcore_barrier()

    @pl.when(cid == 0)
    def _():
        pltpu.sync_copy(acc.at[rslice], out0.at[rslice])

    @pl.when(cid != 0)
    def _():
        pltpu.sync_copy(acc.at[rslice], out1.at[rslice])


def _row_mask(u):
    rows = lax.broadcasted_iota(jnp.int32, (N_PAD, D), 0)
    return jnp.where(rows < N_NODES, u, 0.0)


def _colnorm(u):
    n = jnp.sqrt(jnp.sum(u * u, axis=0, keepdims=True))
    return u / jnp.maximum(n, 1e-12)


def _norm1_body(x_ref, o_ref):
    o_ref[...] = _colnorm(_row_mask(x_ref[...]))


def _norm2_body(a_ref, b_ref, o_ref):
    o_ref[...] = _colnorm(_row_mask(a_ref[...] + b_ref[...]))


_norm1 = pl.pallas_call(
    _norm1_body, out_shape=jax.ShapeDtypeStruct((N_PAD, D), jnp.float32))
_norm2 = pl.pallas_call(
    _norm2_body, out_shape=jax.ShapeDtypeStruct((N_PAD, D), jnp.float32))


def _final_body(a_ref, b_ref, w1_ref, b1_ref, w2_ref, b2_ref, w3_ref,
                b3_ref, o_ref):
    t = _colnorm(_row_mask(a_ref[...] + b_ref[...]))[:N_NODES]
    t = jnp.maximum(
        jnp.dot(t, w1_ref[...], preferred_element_type=jnp.float32)
        + b1_ref[...], 0.0)
    t = jnp.maximum(
        jnp.dot(t, w2_ref[...], preferred_element_type=jnp.float32)
        + b2_ref[...], 0.0)
    o_ref[...] = (
        jnp.dot(t, w3_ref[...], preferred_element_type=jnp.float32)
        + b3_ref[...])


_final = pl.pallas_call(
    _final_body,
    out_shape=jax.ShapeDtypeStruct((N_NODES, D), jnp.float32),
)


def kernel(x, edge_index, W1, b1, W2, b2, W3, b3):
    ei = edge_index.astype(jnp.int32)
    n_extra = E_PAD - E
    pad_iota = jnp.arange(n_extra, dtype=jnp.int32)
    pad_rows = N_NODES + pad_iota % (N_PAD - N_NODES)
    pad_cols = (pad_iota * 13) % N_NODES
    rows = jnp.concatenate([ei[0], pad_rows]).reshape(NW, NCH, 1, CHUNK)
    cols = jnp.concatenate([ei[1], pad_cols]).reshape(NW, NCH, 1, CHUNK)
    idx = jnp.concatenate([rows, cols], axis=2)
    zeros = jnp.zeros((N_PAD, D), jnp.float32)

    h = _norm1(jnp.pad(x[0], ((0, N_PAD - N_NODES), (0, 0))))
    for _ in range(ITRS - 1):
        a0, a1 = _sc_spmm(h, idx, zeros)
        h = _norm2(a0, a1)
    a0, a1 = _sc_spmm(h, idx, zeros)
    out = _final(a0, a1, W1, b1.reshape(1, D), W2, b2.reshape(1, D),
                 W3, b3.reshape(1, D))
    return out[None]


# speedup vs baseline: 11.8449x; 1.0058x over previous
import functools

import jax
import jax.numpy as jnp
from jax import lax
from jax.experimental import pallas as pl
from jax.experimental.pallas import tpu as pltpu
from jax.experimental.pallas import tpu_sc as plsc

N_NODES = 10000
D = 128
E = 320000
ITRS = 5

NW = 32
CHUNK = 64
NCH = 160
EPW = NCH * CHUNK
E_PAD = NW * EPW
N_PAD = 10240
RPS = N_PAD // 16

NSLOT = 3
ISLOT = 6

_mesh = plsc.VectorSubcoreMesh(core_axis_name="c", subcore_axis_name="s")


@functools.partial(
    pl.kernel,
    out_type=(
        jax.ShapeDtypeStruct((N_PAD, D), jnp.float32),
        jax.ShapeDtypeStruct((N_PAD, D), jnp.float32),
    ),
    mesh=_mesh,
    scratch_types=[
        pltpu.VMEM((ISLOT, 2, CHUNK), jnp.int32),
        pltpu.VMEM((NSLOT * CHUNK, D), jnp.float32),
        pltpu.VMEM_SHARED((N_PAD, D), jnp.float32),
        [pltpu.SemaphoreType.DMA] * NSLOT,
        [pltpu.SemaphoreType.DMA] * ISLOT,
    ],
)
def _sc_spmm(h_hbm, idx_hbm, zeros_hbm, out0, out1,
             idx_v, buf, acc, gsems, isems):
    cid = lax.axis_index("c")
    sid = lax.axis_index("s")
    wid = sid * 2 + cid
    rslice = pl.ds(sid * RPS, RPS)

    @pl.when(cid == 0)
    def _():
        pltpu.sync_copy(h_hbm.at[rslice], acc.at[rslice])

    @pl.when(cid != 0)
    def _():
        pltpu.sync_copy(zeros_hbm.at[rslice], acc.at[rslice])

    def bslot(b):
        return buf.at[pl.ds(b * CHUNK, CHUNK)]

    def load_idx(j, s):
        pltpu.async_copy(idx_hbm.at[wid, j], idx_v.at[s], isems[s])

    def wait_idx(j, s):
        pltpu.make_async_copy(idx_hbm.at[wid, j], idx_v.at[s],
                              isems[s]).wait()

    def issue_gather(j, b, s):
        pltpu.async_copy(h_hbm.at[idx_v.at[s, 1]], bslot(b), gsems[b])

    def wait_gather(j, b, s):
        pltpu.make_async_copy(h_hbm.at[idx_v.at[s, 1]], bslot(b),
                              gsems[b]).wait()

    def sync_scatter(j, b, s):
        pltpu.sync_copy(bslot(b), acc.at[idx_v.at[s, 0]], add=True)

    def step(j):
        b, s = j % NSLOT, j % ISLOT
        wait_gather(j, b, s)
        sync_scatter(j, b, s)
        if j + 4 < NCH:
            load_idx(j + 4, (j + 4) % ISLOT)
        if j + 3 < NCH:
            wait_idx(j + 3, (j + 3) % ISLOT)
            issue_gather(j + 3, b, (j + 3) % ISLOT)

    for k in range(4):
        load_idx(k, k)
    plsc.subcore_barrier()
    for k in (0, 1, 2):
        wait_idx(k, k)
        issue_gather(k, k, k)
    step(0)
    step(1)

    def body(g, carry):
        for i in range(6):
            j = 2 + g * 6 + i
            b, s = (2 + i) % NSLOT, (2 + i) % ISLOT
            wait_gather(j, b, s)
            sync_scatter(j, b, s)
            load_idx(j + 4, i % ISLOT)
            wait_idx(j + 3, (5 + i) % ISLOT)
            issue_gather(j + 3, b, (5 + i) % ISLOT)
        return carry

    lax.fori_loop(0, (NCH - 10) // 6, body, 0, unroll=False)
    for j in range(NCH - 8, NCH):
        step(j)

    plsc.subcore_barrier()

    @pl.when(cid == 0)
    def _():
        pltpu.sync_copy(acc.at[rslice], out0.at[rslice])

    @pl.when(cid != 0)
    def _():
        pltpu.sync_copy(acc.at[rslice], out1.at[rslice])


def _row_mask(u):
    rows = lax.broadcasted_iota(jnp.int32, (N_PAD, D), 0)
    return jnp.where(rows < N_NODES, u, 0.0)


def _colnorm(u):
    n = jnp.sqrt(jnp.sum(u * u, axis=0, keepdims=True))
    return u / jnp.maximum(n, 1e-12)


def _norm1_body(x_ref, o_ref):
    o_ref[...] = _colnorm(_row_mask(x_ref[...]))


def _norm2_body(a_ref, b_ref, o_ref):
    o_ref[...] = _colnorm(_row_mask(a_ref[...] + b_ref[...]))


_norm1 = pl.pallas_call(
    _norm1_body, out_shape=jax.ShapeDtypeStruct((N_PAD, D), jnp.float32))
_norm2 = pl.pallas_call(
    _norm2_body, out_shape=jax.ShapeDtypeStruct((N_PAD, D), jnp.float32))


def _final_body(a_ref, b_ref, w1_ref, b1_ref, w2_ref, b2_ref, w3_ref,
                b3_ref, o_ref):
    t = _colnorm(_row_mask(a_ref[...] + b_ref[...]))[:N_NODES]
    t = jnp.maximum(
        jnp.dot(t, w1_ref[...], preferred_element_type=jnp.float32)
        + b1_ref[...], 0.0)
    t = jnp.maximum(
        jnp.dot(t, w2_ref[...], preferred_element_type=jnp.float32)
        + b2_ref[...], 0.0)
    o_ref[...] = (
        jnp.dot(t, w3_ref[...], preferred_element_type=jnp.float32)
        + b3_ref[...])


_final = pl.pallas_call(
    _final_body,
    out_shape=jax.ShapeDtypeStruct((N_NODES, D), jnp.float32),
)


def kernel(x, edge_index, W1, b1, W2, b2, W3, b3):
    ei = edge_index.astype(jnp.int32)
    n_extra = E_PAD - E
    pad_iota = jnp.arange(n_extra, dtype=jnp.int32)
    pad_rows = N_NODES + pad_iota % (N_PAD - N_NODES)
    pad_cols = (pad_iota * 13) % N_NODES
    rows = jnp.concatenate([ei[0], pad_rows]).reshape(NW, NCH, 1, CHUNK)
    cols = jnp.concatenate([ei[1], pad_cols]).reshape(NW, NCH, 1, CHUNK)
    idx = jnp.concatenate([rows, cols], axis=2)
    zeros = jnp.zeros((N_PAD, D), jnp.float32)

    h = _norm1(jnp.pad(x[0], ((0, N_PAD - N_NODES), (0, 0))))
    for _ in range(ITRS - 1):
        a0, a1 = _sc_spmm(h, idx, zeros)
        h = _norm2(a0, a1)
    a0, a1 = _sc_spmm(h, idx, zeros)
    out = _final(a0, a1, W1, b1.reshape(1, D), W2, b2.reshape(1, D),
                 W3, b3.reshape(1, D))
    return out[None]


# speedup vs baseline: 11.9696x; 1.0105x over previous
import functools

import jax
import jax.numpy as jnp
from jax import lax
from jax.experimental import pallas as pl
from jax.experimental.pallas import tpu as pltpu
from jax.experimental.pallas import tpu_sc as plsc

N_NODES = 10000
D = 128
E = 320000
ITRS = 5

NW = 32
CHUNK = 64
NCH = 160
EPW = NCH * CHUNK
E_PAD = NW * EPW
N_PAD = 10240
RPS = N_PAD // 16

NSLOT = 3
ISLOT = 6

_mesh = plsc.VectorSubcoreMesh(core_axis_name="c", subcore_axis_name="s")


@functools.partial(
    pl.kernel,
    out_type=(
        jax.ShapeDtypeStruct((N_PAD, D), jnp.float32),
        jax.ShapeDtypeStruct((N_PAD, D), jnp.float32),
    ),
    mesh=_mesh,
    scratch_types=[
        pltpu.VMEM((ISLOT, 2, CHUNK), jnp.int32),
        pltpu.VMEM((NSLOT * CHUNK, D), jnp.float32),
        pltpu.VMEM_SHARED((N_PAD, D), jnp.float32),
        [pltpu.SemaphoreType.DMA] * NSLOT,
        [pltpu.SemaphoreType.DMA] * ISLOT,
    ],
)
def _sc_spmm(h_hbm, idx_hbm, zeros_hbm, out0, out1,
             idx_v, buf, acc, gsems, isems):
    cid = lax.axis_index("c")
    sid = lax.axis_index("s")
    wid = sid * 2 + cid
    rslice = pl.ds(sid * RPS, RPS)

    def bslot(b):
        return buf.at[pl.ds(b * CHUNK, CHUNK)]

    def load_idx(j, s):
        pltpu.async_copy(idx_hbm.at[wid, j], idx_v.at[s], isems[s])

    def wait_idx(j, s):
        pltpu.make_async_copy(idx_hbm.at[wid, j], idx_v.at[s],
                              isems[s]).wait()

    def issue_gather(j, b, s):
        pltpu.async_copy(h_hbm.at[idx_v.at[s, 1]], bslot(b), gsems[b])

    def wait_gather(j, b, s):
        pltpu.make_async_copy(h_hbm.at[idx_v.at[s, 1]], bslot(b),
                              gsems[b]).wait()

    def sync_scatter(j, b, s):
        pltpu.sync_copy(bslot(b), acc.at[idx_v.at[s, 0]], add=True)

    def step(j):
        b, s = j % NSLOT, j % ISLOT
        wait_gather(j, b, s)
        sync_scatter(j, b, s)
        if j + 4 < NCH:
            load_idx(j + 4, (j + 4) % ISLOT)
        if j + 3 < NCH:
            wait_idx(j + 3, (j + 3) % ISLOT)
            issue_gather(j + 3, b, (j + 3) % ISLOT)

    for k in range(4):
        load_idx(k, k)
    for k in (0, 1, 2):
        wait_idx(k, k)
        issue_gather(k, k, k)

    @pl.when(cid == 0)
    def _():
        pltpu.sync_copy(h_hbm.at[rslice], acc.at[rslice])

    @pl.when(cid != 0)
    def _():
        pltpu.sync_copy(zeros_hbm.at[rslice], acc.at[rslice])

    plsc.subcore_barrier()
    step(0)
    step(1)

    def body(g, carry):
        for i in range(6):
            j = 2 + g * 6 + i
            b, s = (2 + i) % NSLOT, (2 + i) % ISLOT
            wait_gather(j, b, s)
            sync_scatter(j, b, s)
            load_idx(j + 4, i % ISLOT)
            wait_idx(j + 3, (5 + i) % ISLOT)
            issue_gather(j + 3, b, (5 + i) % ISLOT)
        return carry

    lax.fori_loop(0, (NCH - 10) // 6, body, 0, unroll=False)
    for j in range(NCH - 8, NCH):
        step(j)

    plsc.subcore_barrier()

    @pl.when(cid == 0)
    def _():
        pltpu.sync_copy(acc.at[rslice], out0.at[rslice])

    @pl.when(cid != 0)
    def _():
        pltpu.sync_copy(acc.at[rslice], out1.at[rslice])


def _row_mask(u):
    rows = lax.broadcasted_iota(jnp.int32, (N_PAD, D), 0)
    return jnp.where(rows < N_NODES, u, 0.0)


def _colnorm(u):
    n = jnp.sqrt(jnp.sum(u * u, axis=0, keepdims=True))
    return u / jnp.maximum(n, 1e-12)


def _norm1_body(x_ref, o_ref):
    o_ref[...] = _colnorm(_row_mask(x_ref[...]))


def _norm2_body(a_ref, b_ref, o_ref):
    o_ref[...] = _colnorm(_row_mask(a_ref[...] + b_ref[...]))


_norm1 = pl.pallas_call(
    _norm1_body, out_shape=jax.ShapeDtypeStruct((N_PAD, D), jnp.float32))
_norm2 = pl.pallas_call(
    _norm2_body, out_shape=jax.ShapeDtypeStruct((N_PAD, D), jnp.float32))


def _final_body(a_ref, b_ref, w1_ref, b1_ref, w2_ref, b2_ref, w3_ref,
                b3_ref, o_ref):
    t = _colnorm(_row_mask(a_ref[...] + b_ref[...]))[:N_NODES]
    t = jnp.maximum(
        jnp.dot(t, w1_ref[...], preferred_element_type=jnp.float32)
        + b1_ref[...], 0.0)
    t = jnp.maximum(
        jnp.dot(t, w2_ref[...], preferred_element_type=jnp.float32)
        + b2_ref[...], 0.0)
    o_ref[...] = (
        jnp.dot(t, w3_ref[...], preferred_element_type=jnp.float32)
        + b3_ref[...])


_final = pl.pallas_call(
    _final_body,
    out_shape=jax.ShapeDtypeStruct((N_NODES, D), jnp.float32),
)


def kernel(x, edge_index, W1, b1, W2, b2, W3, b3):
    ei = edge_index.astype(jnp.int32)
    n_extra = E_PAD - E
    pad_iota = jnp.arange(n_extra, dtype=jnp.int32)
    pad_rows = N_NODES + pad_iota % (N_PAD - N_NODES)
    pad_cols = (pad_iota * 13) % N_NODES
    rows = jnp.concatenate([ei[0], pad_rows]).reshape(NW, NCH, 1, CHUNK)
    cols = jnp.concatenate([ei[1], pad_cols]).reshape(NW, NCH, 1, CHUNK)
    idx = jnp.concatenate([rows, cols], axis=2)
    zeros = jnp.zeros((N_PAD, D), jnp.float32)

    h = _norm1(jnp.pad(x[0], ((0, N_PAD - N_NODES), (0, 0))))
    for _ in range(ITRS - 1):
        a0, a1 = _sc_spmm(h, idx, zeros)
        h = _norm2(a0, a1)
    a0, a1 = _sc_spmm(h, idx, zeros)
    out = _final(a0, a1, W1, b1.reshape(1, D), W2, b2.reshape(1, D),
                 W3, b3.reshape(1, D))
    return out[None]


# speedup vs baseline: 12.0405x; 1.0059x over previous
import functools

import jax
import jax.numpy as jnp
from jax import lax
from jax.experimental import pallas as pl
from jax.experimental.pallas import tpu as pltpu
from jax.experimental.pallas import tpu_sc as plsc

N_NODES = 10000
D = 128
E = 320000
ITRS = 5

NW = 32
CHUNK = 128
NCH = 80
EPW = NCH * CHUNK
E_PAD = NW * EPW
N_PAD = 10240
RPS = N_PAD // 16

NSLOT = 2
ISLOT = 4

_mesh = plsc.VectorSubcoreMesh(core_axis_name="c", subcore_axis_name="s")


@functools.partial(
    pl.kernel,
    out_type=(
        jax.ShapeDtypeStruct((N_PAD, D), jnp.float32),
        jax.ShapeDtypeStruct((N_PAD, D), jnp.float32),
    ),
    mesh=_mesh,
    scratch_types=[
        pltpu.VMEM((ISLOT, 2, CHUNK), jnp.int32),
        pltpu.VMEM((NSLOT * CHUNK, D), jnp.float32),
        pltpu.VMEM_SHARED((N_PAD, D), jnp.float32),
        [pltpu.SemaphoreType.DMA] * NSLOT,
        [pltpu.SemaphoreType.DMA] * ISLOT,
    ],
)
def _sc_spmm(h_hbm, idx_hbm, zeros_hbm, out0, out1,
             idx_v, buf, acc, gsems, isems):
    cid = lax.axis_index("c")
    sid = lax.axis_index("s")
    wid = sid * 2 + cid
    rslice = pl.ds(sid * RPS, RPS)

    def bslot(b):
        return buf.at[pl.ds(b * CHUNK, CHUNK)]

    def load_idx(j, s):
        pltpu.async_copy(idx_hbm.at[0, wid, j], idx_v.at[s, 0], isems[s])
        pltpu.async_copy(idx_hbm.at[1, wid, j], idx_v.at[s, 1], isems[s])

    def wait_idx(j, s):
        pltpu.make_async_copy(idx_hbm.at[0, wid, j], idx_v.at[s, 0],
                              isems[s]).wait()
        pltpu.make_async_copy(idx_hbm.at[1, wid, j], idx_v.at[s, 1],
                              isems[s]).wait()

    def issue_gather(j, b, s):
        pltpu.async_copy(h_hbm.at[idx_v.at[s, 1]], bslot(b), gsems[b])

    def wait_gather(j, b, s):
        pltpu.make_async_copy(h_hbm.at[idx_v.at[s, 1]], bslot(b),
                              gsems[b]).wait()

    def sync_scatter(j, b, s):
        pltpu.sync_copy(bslot(b), acc.at[idx_v.at[s, 0]], add=True)

    def step(j):
        b, s = j % NSLOT, j % ISLOT
        wait_gather(j, b, s)
        sync_scatter(j, b, s)
        if j + 3 < NCH:
            load_idx(j + 3, (j + 3) % ISLOT)
        if j + 2 < NCH:
            wait_idx(j + 2, (j + 2) % ISLOT)
            issue_gather(j + 2, b, (j + 2) % ISLOT)

    for k in range(3):
        load_idx(k, k)
    for k in (0, 1):
        wait_idx(k, k)
        issue_gather(k, k, k)

    @pl.when(cid == 0)
    def _():
        pltpu.sync_copy(h_hbm.at[rslice], acc.at[rslice])

    @pl.when(cid != 0)
    def _():
        pltpu.sync_copy(zeros_hbm.at[rslice], acc.at[rslice])

    plsc.subcore_barrier()
    step(0)

    def body(g, carry):
        for i in range(4):
            j = 1 + g * 4 + i
            b, s = (1 + i) % NSLOT, (1 + i) % ISLOT
            wait_gather(j, b, s)
            sync_scatter(j, b, s)
            load_idx(j + 3, i % ISLOT)
            wait_idx(j + 2, (3 + i) % ISLOT)
            issue_gather(j + 2, b, (3 + i) % ISLOT)
        return carry

    lax.fori_loop(0, (NCH - 4) // 4, body, 0, unroll=False)
    for j in range(NCH - 3, NCH):
        step(j)

    plsc.subcore_barrier()

    @pl.when(cid == 0)
    def _():
        pltpu.sync_copy(acc.at[rslice], out0.at[rslice])

    @pl.when(cid != 0)
    def _():
        pltpu.sync_copy(acc.at[rslice], out1.at[rslice])


def _row_mask(u):
    rows = lax.broadcasted_iota(jnp.int32, (N_PAD, D), 0)
    return jnp.where(rows < N_NODES, u, 0.0)


def _colnorm(u):
    n = jnp.sqrt(jnp.sum(u * u, axis=0, keepdims=True))
    return u / jnp.maximum(n, 1e-12)


def _norm1_body(x_ref, o_ref):
    o_ref[...] = _colnorm(_row_mask(x_ref[...]))


def _norm2_body(a_ref, b_ref, o_ref):
    o_ref[...] = _colnorm(_row_mask(a_ref[...] + b_ref[...]))


_norm1 = pl.pallas_call(
    _norm1_body, out_shape=jax.ShapeDtypeStruct((N_PAD, D), jnp.float32))
_norm2 = pl.pallas_call(
    _norm2_body, out_shape=jax.ShapeDtypeStruct((N_PAD, D), jnp.float32))


def _final_body(a_ref, b_ref, w1_ref, b1_ref, w2_ref, b2_ref, w3_ref,
                b3_ref, o_ref):
    t = _colnorm(_row_mask(a_ref[...] + b_ref[...]))[:N_NODES]
    t = jnp.maximum(
        jnp.dot(t, w1_ref[...], preferred_element_type=jnp.float32)
        + b1_ref[...], 0.0)
    t = jnp.maximum(
        jnp.dot(t, w2_ref[...], preferred_element_type=jnp.float32)
        + b2_ref[...], 0.0)
    o_ref[...] = (
        jnp.dot(t, w3_ref[...], preferred_element_type=jnp.float32)
        + b3_ref[...])


_final = pl.pallas_call(
    _final_body,
    out_shape=jax.ShapeDtypeStruct((N_NODES, D), jnp.float32),
)


def kernel(x, edge_index, W1, b1, W2, b2, W3, b3):
    ei = edge_index.astype(jnp.int32)
    n_extra = E_PAD - E
    pad_iota = jnp.arange(n_extra, dtype=jnp.int32)
    pad_rows = N_NODES + pad_iota % (N_PAD - N_NODES)
    pad_cols = (pad_iota * 13) % N_NODES
    idx = jnp.concatenate(
        [ei[0], pad_rows, ei[1], pad_cols]).reshape(2, NW, NCH, CHUNK)
    zeros = jnp.zeros((N_PAD, D), jnp.float32)

    h = _norm1(jnp.pad(x[0], ((0, N_PAD - N_NODES), (0, 0))))
    for _ in range(ITRS - 1):
        a0, a1 = _sc_spmm(h, idx, zeros)
        h = _norm2(a0, a1)
    a0, a1 = _sc_spmm(h, idx, zeros)
    out = _final(a0, a1, W1, b1.reshape(1, D), W2, b2.reshape(1, D),
                 W3, b3.reshape(1, D))
    return out[None]


# speedup vs baseline: 12.0738x; 1.0028x over previous
import functools

import jax
import jax.numpy as jnp
from jax import lax
from jax.experimental import pallas as pl
from jax.experimental.pallas import tpu as pltpu
from jax.experimental.pallas import tpu_sc as plsc

N_NODES = 10000
D = 128
E = 320000
ITRS = 5

NW = 32
CHUNK = 128
NCH = 80
EPW = NCH * CHUNK
E_PAD = NW * EPW
N_PAD = 10240
RPS = N_PAD // 16

NSLOT = 2
ISLOT = 4

_mesh = plsc.VectorSubcoreMesh(core_axis_name="c", subcore_axis_name="s")


@functools.partial(
    pl.kernel,
    out_type=(
        jax.ShapeDtypeStruct((N_PAD, D), jnp.float32),
        jax.ShapeDtypeStruct((N_PAD, D), jnp.float32),
    ),
    mesh=_mesh,
    scratch_types=[
        pltpu.VMEM((ISLOT, 2, CHUNK), jnp.int32),
        pltpu.VMEM((NSLOT * CHUNK, D), jnp.float32),
        pltpu.VMEM_SHARED((N_PAD, D), jnp.float32),
        [pltpu.SemaphoreType.DMA] * NSLOT,
        [pltpu.SemaphoreType.DMA] * ISLOT,
    ],
)
def _sc_spmm(h_hbm, idx_hbm, zeros_hbm, out0, out1,
             idx_v, buf, acc, gsems, isems):
    cid = lax.axis_index("c")
    sid = lax.axis_index("s")
    wid = sid * 2 + cid
    rslice = pl.ds(sid * RPS, RPS)

    def bslot(b):
        return buf.at[pl.ds(b * CHUNK, CHUNK)]

    def load_idx(j, s):
        pltpu.async_copy(idx_hbm.at[0, wid, j], idx_v.at[s, 0], isems[s])
        pltpu.async_copy(idx_hbm.at[1, wid, j], idx_v.at[s, 1], isems[s])

    def wait_idx(j, s):
        pltpu.make_async_copy(idx_hbm.at[0, wid, j], idx_v.at[s, 0],
                              isems[s]).wait()
        pltpu.make_async_copy(idx_hbm.at[1, wid, j], idx_v.at[s, 1],
                              isems[s]).wait()

    def issue_gather(j, b, s):
        pltpu.async_copy(h_hbm.at[idx_v.at[s, 1]], bslot(b), gsems[b])

    def wait_gather(j, b, s):
        pltpu.make_async_copy(h_hbm.at[idx_v.at[s, 1]], bslot(b),
                              gsems[b]).wait()

    def sync_scatter(j, b, s):
        pltpu.sync_copy(bslot(b), acc.at[idx_v.at[s, 0]], add=True)

    def step(j):
        b, s = j % NSLOT, j % ISLOT
        wait_gather(j, b, s)
        sync_scatter(j, b, s)
        if j + 3 < NCH:
            load_idx(j + 3, (j + 3) % ISLOT)
        if j + 2 < NCH:
            wait_idx(j + 2, (j + 2) % ISLOT)
            issue_gather(j + 2, b, (j + 2) % ISLOT)

    for k in range(3):
        load_idx(k, k)
    for k in (0, 1):
        wait_idx(k, k)
        issue_gather(k, k, k)

    @pl.when(cid == 0)
    def _():
        pltpu.sync_copy(h_hbm.at[rslice], acc.at[rslice])

    @pl.when(cid != 0)
    def _():
        pltpu.sync_copy(zeros_hbm.at[rslice], acc.at[rslice])

    plsc.subcore_barrier()
    step(0)

    def body(g, carry):
        for i in range(4):
            j = 1 + g * 4 + i
            b, s = (1 + i) % NSLOT, (1 + i) % ISLOT
            wait_gather(j, b, s)
            sync_scatter(j, b, s)
            load_idx(j + 3, i % ISLOT)
            wait_idx(j + 2, (3 + i) % ISLOT)
            issue_gather(j + 2, b, (3 + i) % ISLOT)
        return carry

    lax.fori_loop(0, (NCH - 4) // 4, body, 0, unroll=False)
    for j in range(NCH - 3, NCH):
        step(j)

    plsc.subcore_barrier()

    @pl.when(cid == 0)
    def _():
        pltpu.sync_copy(acc.at[rslice], out0.at[rslice])

    @pl.when(cid != 0)
    def _():
        pltpu.sync_copy(acc.at[rslice], out1.at[rslice])


def _row_mask(u):
    rows = lax.broadcasted_iota(jnp.int32, (N_PAD, D), 0)
    return jnp.where(rows < N_NODES, u, 0.0)


def _colnorm(u):
    n = jnp.sqrt(jnp.sum(u * u, axis=0, keepdims=True))
    return u / jnp.maximum(n, 1e-12)


def _norm1_body(x_ref, o_ref):
    u = x_ref[0]
    n = jnp.sqrt(jnp.sum(u * u, axis=0, keepdims=True))
    o_ref[:N_NODES] = u / jnp.maximum(n, 1e-12)
    o_ref[N_NODES:] = jnp.zeros((N_PAD - N_NODES, D), jnp.float32)


def _norm2_body(a_ref, b_ref, o_ref):
    o_ref[...] = _colnorm(_row_mask(a_ref[...] + b_ref[...]))


_norm1 = pl.pallas_call(
    _norm1_body, out_shape=jax.ShapeDtypeStruct((N_PAD, D), jnp.float32))
_norm2 = pl.pallas_call(
    _norm2_body, out_shape=jax.ShapeDtypeStruct((N_PAD, D), jnp.float32))


def _final_body(a_ref, b_ref, w1_ref, b1_ref, w2_ref, b2_ref, w3_ref,
                b3_ref, o_ref):
    t = _colnorm(_row_mask(a_ref[...] + b_ref[...]))[:N_NODES]
    t = jnp.maximum(
        jnp.dot(t, w1_ref[...], preferred_element_type=jnp.float32)
        + b1_ref[...], 0.0)
    t = jnp.maximum(
        jnp.dot(t, w2_ref[...], preferred_element_type=jnp.float32)
        + b2_ref[...], 0.0)
    o_ref[...] = (
        jnp.dot(t, w3_ref[...], preferred_element_type=jnp.float32)
        + b3_ref[...])


_final = pl.pallas_call(
    _final_body,
    out_shape=jax.ShapeDtypeStruct((N_NODES, D), jnp.float32),
)


def kernel(x, edge_index, W1, b1, W2, b2, W3, b3):
    ei = edge_index.astype(jnp.int32)
    n_extra = E_PAD - E
    pad_iota = jnp.arange(n_extra, dtype=jnp.int32)
    pad_rows = N_NODES + pad_iota % (N_PAD - N_NODES)
    pad_cols = (pad_iota * 13) % N_NODES
    idx = jnp.concatenate(
        [ei[0], pad_rows, ei[1], pad_cols]).reshape(2, NW, NCH, CHUNK)
    zeros = jnp.zeros((N_PAD, D), jnp.float32)

    h = _norm1(x)
    for _ in range(ITRS - 1):
        a0, a1 = _sc_spmm(h, idx, zeros)
        h = _norm2(a0, a1)
    a0, a1 = _sc_spmm(h, idx, zeros)
    out = _final(a0, a1, W1, b1.reshape(1, D), W2, b2.reshape(1, D),
                 W3, b3.reshape(1, D))
    return out[None]
